# baseline (speedup 1.0000x reference)
import functools

import jax
import jax.numpy as jnp
from jax import lax
from jax.experimental import pallas as pl
from jax.experimental.pallas import tpu as pltpu
from jax.experimental.pallas import tpu_sc as plsc

_NC = 2
_NS = 16
_NW = _NC * _NS


def _tc_pack(wT):
    D, N = wT.shape
    grid = (N + 1023) // 1024

    def body(in_ref, o_ref):
        x = in_ref[...]
        o_ref[:, :D] = x[:, :512].T
        o_ref[:, D:] = x[:, 512:].T

    return pl.pallas_call(
        body,
        grid=(grid,),
        in_specs=[pl.BlockSpec((D, 1024), lambda i: (0, i))],
        out_specs=pl.BlockSpec((512, 2 * D), lambda i: (i, 0)),
        out_shape=jax.ShapeDtypeStruct((grid * 512, 2 * D), jnp.float32),
        compiler_params=pltpu.CompilerParams(
            dimension_semantics=("parallel",)),
    )(wT)


def _sc_gather(h2, r2, t2, ewP, rwP):
    W = ewP.shape[1]
    B = h2.shape[0]
    bpw = B // _NW
    hbpw = bpw // 2
    mesh = plsc.VectorSubcoreMesh(core_axis_name="c", subcore_axis_name="s")
    row_t = jax.ShapeDtypeStruct((B, W), jnp.float32)

    @functools.partial(
        pl.kernel,
        mesh=mesh,
        out_type=[row_t, row_t, row_t],
        scratch_types=[
            pltpu.VMEM((bpw,), jnp.int32),
            pltpu.VMEM((bpw,), jnp.int32),
            pltpu.VMEM((bpw,), jnp.int32),
            pltpu.VMEM((hbpw, W), jnp.float32),
            pltpu.VMEM((hbpw, W), jnp.float32),
            pltpu.VMEM((hbpw, W), jnp.float32),
            pltpu.SemaphoreType.DMA,
            pltpu.SemaphoreType.DMA,
            pltpu.SemaphoreType.DMA,
        ],
    )
    def k(h_hbm, r_hbm, t_hbm, ew_hbm, rw_hbm, oh_hbm, or_hbm, ot_hbm,
          hi_v, ri_v, ti_v, hr_v, rr_v, tr_v, sem_h, sem_r, sem_t):
        wid = lax.axis_index("s") * _NC + lax.axis_index("c")
        base = wid * bpw
        pltpu.sync_copy(h_hbm.at[pl.ds(base, bpw)], hi_v)
        pltpu.sync_copy(t_hbm.at[pl.ds(base, bpw)], ti_v)
        pltpu.sync_copy(r_hbm.at[pl.ds(base, bpw)], ri_v)
        for half in range(2):
            off = half * hbpw
            ch = pltpu.async_copy(
                ew_hbm.at[hi_v.at[pl.ds(off, hbpw)]], hr_v, sem_h)
            ct = pltpu.async_copy(
                ew_hbm.at[ti_v.at[pl.ds(off, hbpw)]], tr_v, sem_t)
            cr = pltpu.async_copy(
                rw_hbm.at[ri_v.at[pl.ds(off, hbpw)]], rr_v, sem_r)
            ch.wait()
            pltpu.sync_copy(hr_v, oh_hbm.at[pl.ds(base + off, hbpw)])
            ct.wait()
            pltpu.sync_copy(tr_v, ot_hbm.at[pl.ds(base + off, hbpw)])
            cr.wait()
            pltpu.sync_copy(rr_v, or_hbm.at[pl.ds(base + off, hbpw)])

    return k(h2, r2, t2, ewP, rwP)


def _tc_score(h2, r2, t2, hp, rp, tp, max_norm=1.0):
    B, W = h2.shape
    D = W // 2
    blk = 2048

    def body(h_ref, r_ref, t_ref, hp_ref, rp_ref, tp_ref, o_ref):
        def pick(x_ref, p_ref):
            x = x_ref[...]
            p = p_ref[...][:, None] > 0
            return jnp.where(p, x[:, D:], x[:, :D])

        hv = pick(h_ref, hp_ref)
        rv = pick(r_ref, rp_ref)
        tv = pick(t_ref, tp_ref)
        nh = jnp.sqrt(jnp.sum(hv * hv, axis=1, keepdims=True))
        sh = jnp.where(nh > max_norm, max_norm / (nh + 1e-7), 1.0)
        nt = jnp.sqrt(jnp.sum(tv * tv, axis=1, keepdims=True))
        st = jnp.where(nt > max_norm, max_norm / (nt + 1e-7), 1.0)
        d = hv * sh + rv - tv * st
        o_ref[...] = -jnp.sqrt(jnp.sum(d * d, axis=1))

    vspec = pl.BlockSpec((blk, W), lambda i: (i, 0))
    pspec = pl.BlockSpec((blk,), lambda i: (i,))
    return pl.pallas_call(
        body,
        grid=(B // blk,),
        in_specs=[vspec, vspec, vspec, pspec, pspec, pspec],
        out_specs=pspec,
        out_shape=jax.ShapeDtypeStruct((B,), jnp.float32),
        compiler_params=pltpu.CompilerParams(
            dimension_semantics=("parallel",)),
    )(h2, r2, t2, hp, rp, tp)


def kernel(heads, relations, tails, entityW, relationW):
    heads = heads.astype(jnp.int32)
    relations = relations.astype(jnp.int32)
    tails = tails.astype(jnp.int32)
    ewP = _tc_pack(entityW.T)
    rwP = _tc_pack(relationW.T)

    def prow(i):
        return ((i >> 10) << 9) + (i & 511)

    h2, r2, t2 = _sc_gather(prow(heads), prow(relations), prow(tails),
                            ewP, rwP)
    return _tc_score(h2, r2, t2,
                     (heads >> 9) & 1, (relations >> 9) & 1,
                     (tails >> 9) & 1)


# speedup vs baseline: 1.1356x; 1.1356x over previous
import functools

import jax
import jax.numpy as jnp
from jax import lax
from jax.experimental import pallas as pl
from jax.experimental.pallas import tpu as pltpu
from jax.experimental.pallas import tpu_sc as plsc

_NC = 2
_NS = 16
_NW = _NC * _NS


def _tc_pack(wT):
    D, N = wT.shape
    grid = (N + 1023) // 1024

    def body(in_ref, o_ref):
        x = in_ref[...]
        o_ref[:, :D] = x[:, :512].T
        o_ref[:, D:] = x[:, 512:].T

    return pl.pallas_call(
        body,
        grid=(grid,),
        in_specs=[pl.BlockSpec((D, 1024), lambda i: (0, i))],
        out_specs=pl.BlockSpec((512, 2 * D), lambda i: (i, 0)),
        out_shape=jax.ShapeDtypeStruct((grid * 512, 2 * D), jnp.float32),
        compiler_params=pltpu.CompilerParams(
            dimension_semantics=("parallel",)),
    )(wT)


def _sc_gather(h2, r2, t2, ewP, rwP):
    W = ewP.shape[1]
    B = h2.shape[0]
    bpw = B // _NW
    hbpw = bpw // 2
    mesh = plsc.VectorSubcoreMesh(core_axis_name="c", subcore_axis_name="s")
    row_t = jax.ShapeDtypeStruct((B, W), jnp.float32)

    @functools.partial(
        pl.kernel,
        mesh=mesh,
        out_type=[row_t, row_t, row_t],
        scratch_types=[
            pltpu.VMEM((bpw,), jnp.int32),
            pltpu.VMEM((bpw,), jnp.int32),
            pltpu.VMEM((bpw,), jnp.int32),
            pltpu.VMEM((hbpw, W), jnp.float32),
            pltpu.VMEM((hbpw, W), jnp.float32),
            pltpu.VMEM((hbpw, W), jnp.float32),
            pltpu.SemaphoreType.DMA,
            pltpu.SemaphoreType.DMA,
            pltpu.SemaphoreType.DMA,
        ],
    )
    def k(h_hbm, r_hbm, t_hbm, ew_hbm, rw_hbm, oh_hbm, or_hbm, ot_hbm,
          hi_v, ri_v, ti_v, hr_v, rr_v, tr_v, sem_h, sem_r, sem_t):
        wid = lax.axis_index("s") * _NC + lax.axis_index("c")
        base = wid * bpw
        pltpu.sync_copy(h_hbm.at[pl.ds(base, bpw)], hi_v)
        pltpu.sync_copy(t_hbm.at[pl.ds(base, bpw)], ti_v)
        pltpu.sync_copy(r_hbm.at[pl.ds(base, bpw)], ri_v)
        for half in range(2):
            off = half * hbpw
            ch = pltpu.async_copy(
                ew_hbm.at[hi_v.at[pl.ds(off, hbpw)]], hr_v, sem_h)
            ct = pltpu.async_copy(
                ew_hbm.at[ti_v.at[pl.ds(off, hbpw)]], tr_v, sem_t)
            cr = pltpu.async_copy(
                rw_hbm.at[ri_v.at[pl.ds(off, hbpw)]], rr_v, sem_r)
            ch.wait()
            pltpu.sync_copy(hr_v, oh_hbm.at[pl.ds(base + off, hbpw)])
            ct.wait()
            pltpu.sync_copy(tr_v, ot_hbm.at[pl.ds(base + off, hbpw)])
            cr.wait()
            pltpu.sync_copy(rr_v, or_hbm.at[pl.ds(base + off, hbpw)])

    return k(h2, r2, t2, ewP, rwP)


def _tc_score(h2, r2, t2, hp, rp, tp, max_norm=1.0):
    B, W = h2.shape
    D = W // 2
    blk = 2048

    def body(h_ref, r_ref, t_ref, hp_ref, rp_ref, tp_ref, o_ref):
        def pick(x_ref, p_ref):
            x = x_ref[...]
            p = p_ref[...][:, None] > 0
            return jnp.where(p, x[:, D:], x[:, :D])

        hv = pick(h_ref, hp_ref)
        rv = pick(r_ref, rp_ref)
        tv = pick(t_ref, tp_ref)
        nh = jnp.sqrt(jnp.sum(hv * hv, axis=1, keepdims=True))
        sh = jnp.where(nh > max_norm, max_norm / (nh + 1e-7), 1.0)
        nt = jnp.sqrt(jnp.sum(tv * tv, axis=1, keepdims=True))
        st = jnp.where(nt > max_norm, max_norm / (nt + 1e-7), 1.0)
        d = hv * sh + rv - tv * st
        o_ref[...] = -jnp.sqrt(jnp.sum(d * d, axis=1))

    vspec = pl.BlockSpec((blk, W), lambda i: (i, 0))
    pspec = pl.BlockSpec((blk,), lambda i: (i,))
    return pl.pallas_call(
        body,
        grid=(B // blk,),
        in_specs=[vspec, vspec, vspec, pspec, pspec, pspec],
        out_specs=pspec,
        out_shape=jax.ShapeDtypeStruct((B,), jnp.float32),
        compiler_params=pltpu.CompilerParams(
            dimension_semantics=("parallel",)),
    )(h2, r2, t2, hp, rp, tp)


def kernel(heads, relations, tails, entityW, relationW):
    heads = heads.astype(jnp.int32)
    relations = relations.astype(jnp.int32)
    tails = tails.astype(jnp.int32)
    en, d = entityW.shape
    rn, _ = relationW.shape
    ewP = entityW.reshape(en // 2, 2 * d)
    rwP = relationW.reshape(rn // 2, 2 * d)
    h2, r2, t2 = _sc_gather(heads >> 1, relations >> 1, tails >> 1,
                            ewP, rwP)
    return _tc_score(h2, r2, t2, heads & 1, relations & 1, tails & 1)


# speedup vs baseline: 1.9386x; 1.7070x over previous
import functools

import jax
import jax.numpy as jnp
from jax import lax
from jax.experimental import pallas as pl
from jax.experimental.pallas import tpu as pltpu
from jax.experimental.pallas import tpu_sc as plsc

_NC = 2
_NS = 16
_NW = _NC * _NS


def _sc_gather(heads, relations, tails, entityW, relationW):
    B = heads.shape[0]
    D = entityW.shape[1]
    bpw = B // _NW
    hbpw = bpw // 2
    mesh = plsc.VectorSubcoreMesh(core_axis_name="c", subcore_axis_name="s")
    row_t = jax.ShapeDtypeStruct((B, D), jnp.float32)

    @functools.partial(
        pl.kernel,
        mesh=mesh,
        out_type=[row_t, row_t, row_t],
        scratch_types=[
            pltpu.VMEM((bpw,), jnp.int32),
            pltpu.VMEM((bpw,), jnp.int32),
            pltpu.VMEM((bpw,), jnp.int32),
            pltpu.VMEM((hbpw, D), jnp.float32),
            pltpu.VMEM((hbpw, D), jnp.float32),
            pltpu.VMEM((hbpw, D), jnp.float32),
            pltpu.SemaphoreType.DMA,
            pltpu.SemaphoreType.DMA,
            pltpu.SemaphoreType.DMA,
        ],
    )
    def k(h_hbm, r_hbm, t_hbm, ew_hbm, rw_hbm, oh_hbm, or_hbm, ot_hbm,
          hi_s, ri_s, ti_s, hr_v, rr_v, tr_v, sem_h, sem_r, sem_t):
        wid = lax.axis_index("s") * _NC + lax.axis_index("c")
        base = wid * bpw
        pltpu.sync_copy(h_hbm.at[pl.ds(base, bpw)], hi_s)
        pltpu.sync_copy(t_hbm.at[pl.ds(base, bpw)], ti_s)
        pltpu.sync_copy(r_hbm.at[pl.ds(base, bpw)], ri_s)
        for half in range(2):
            off = half * hbpw

            @pl.loop(0, hbpw // 16)
            def _(g):
                row = off + g * 16
                hv = hi_s[pl.ds(row, 16)]
                tv = ti_s[pl.ds(row, 16)]
                rv = ri_s[pl.ds(row, 16)]
                for j in range(16):
                    dst = g * 16 + j
                    pltpu.async_copy(ew_hbm.at[hv[j]], hr_v.at[dst], sem_h)
                    pltpu.async_copy(ew_hbm.at[tv[j]], tr_v.at[dst], sem_t)
                    pltpu.async_copy(rw_hbm.at[rv[j]], rr_v.at[dst], sem_r)

            pltpu.make_async_copy(oh_hbm.at[pl.ds(0, hbpw)], hr_v, sem_h).wait()
            pltpu.make_async_copy(ot_hbm.at[pl.ds(0, hbpw)], tr_v, sem_t).wait()
            pltpu.make_async_copy(or_hbm.at[pl.ds(0, hbpw)], rr_v, sem_r).wait()
            pltpu.sync_copy(hr_v, oh_hbm.at[pl.ds(base + off, hbpw)])
            pltpu.sync_copy(tr_v, ot_hbm.at[pl.ds(base + off, hbpw)])
            pltpu.sync_copy(rr_v, or_hbm.at[pl.ds(base + off, hbpw)])

    return k(heads, relations, tails, entityW, relationW)


def _tc_score(h, r, t, max_norm=1.0):
    B, D = h.shape
    blk = 2048

    def body(h_ref, r_ref, t_ref, o_ref):
        hv = h_ref[...]
        rv = r_ref[...]
        tv = t_ref[...]
        nh = jnp.sqrt(jnp.sum(hv * hv, axis=1, keepdims=True))
        sh = jnp.where(nh > max_norm, max_norm / (nh + 1e-7), 1.0)
        nt = jnp.sqrt(jnp.sum(tv * tv, axis=1, keepdims=True))
        st = jnp.where(nt > max_norm, max_norm / (nt + 1e-7), 1.0)
        d = hv * sh + rv - tv * st
        o_ref[...] = -jnp.sqrt(jnp.sum(d * d, axis=1))

    return pl.pallas_call(
        body,
        grid=(B // blk,),
        in_specs=[
            pl.BlockSpec((blk, D), lambda i: (i, 0)),
            pl.BlockSpec((blk, D), lambda i: (i, 0)),
            pl.BlockSpec((blk, D), lambda i: (i, 0)),
        ],
        out_specs=pl.BlockSpec((blk,), lambda i: (i,)),
        out_shape=jax.ShapeDtypeStruct((B,), jnp.float32),
    )(h, r, t)


def kernel(heads, relations, tails, entityW, relationW):
    heads = heads.astype(jnp.int32)
    relations = relations.astype(jnp.int32)
    tails = tails.astype(jnp.int32)
    h, r, t = _sc_gather(heads, relations, tails, entityW, relationW)
    return _tc_score(h, r, t)
